# Initial kernel scaffold; baseline (speedup 1.0000x reference)
import jax
import jax.numpy as jnp
from jax.experimental import pallas as pl


def kernel(coordinates, active, occupancies, lmax, radial_densities, grid_to_cartesian):
    raise NotImplementedError("write your pallas kernel here")


# speedup vs baseline: 998.3180x; 998.3180x over previous
import functools

import jax
import jax.numpy as jnp
from jax import lax
from jax.experimental import pallas as pl
from jax.experimental.pallas import tpu as pltpu
from jax.experimental.pallas import tpu_sc as plsc

GRID = 32
RSTEP = 0.1
RMAX = 3.0
NATOMS = 128
NRAD = 64
L = 16


def _splat(vec, j):
  return jnp.full((L,), vec[j], dtype=vec.dtype)


def _sc_body(ax_h, ay_h, az_h, occ_h, zlo_h, zhi_h, ylo_h, ycnt_h, dens_h,
             gv_h, out_h,
             ax_v, ay_v, az_v, occ_v, zlo_v, zhi_v, ylo_v, ycnt_v, dens_v,
             gv_v, plane_v, sem):
  cid = lax.axis_index("c")
  sid = lax.axis_index("s")
  wid = sid * 2 + cid

  pltpu.async_copy(dens_h, dens_v, sem).wait()
  pltpu.async_copy(ax_h, ax_v, sem).wait()
  pltpu.async_copy(ay_h, ay_v, sem).wait()
  pltpu.async_copy(az_h, az_v, sem).wait()
  pltpu.async_copy(occ_h, occ_v, sem).wait()
  pltpu.async_copy(zlo_h, zlo_v, sem).wait()
  pltpu.async_copy(zhi_h, zhi_v, sem).wait()
  pltpu.async_copy(ylo_h, ylo_v, sem).wait()
  pltpu.async_copy(ycnt_h, ycnt_v, sem).wait()
  pltpu.async_copy(gv_h, gv_v, sem).wait()

  gv = gv_v[...]
  g00 = _splat(gv, 0)
  g01 = _splat(gv, 1)
  g02 = _splat(gv, 2)
  g11 = _splat(gv, 3)
  g12 = _splat(gv, 4)
  g22 = _splat(gv, 5)

  zf = jnp.full((L,), wid, dtype=jnp.int32).astype(jnp.float32)
  iota = lax.iota(jnp.int32, L)
  xf0 = iota.astype(jnp.float32)
  xf1 = (iota + 16).astype(jnp.float32)

  zero16 = jnp.zeros((L,), jnp.float32)

  def zero_body(r, _):
    plane_v[r] = zero16
    return _

  lax.fori_loop(0, 2 * GRID, zero_body, None)

  rmax2 = jnp.full((L,), RMAX * RMAX, jnp.float32)
  half = jnp.full((L,), 0.5, jnp.float32)
  three_half = jnp.full((L,), 1.5, jnp.float32)
  magic = jnp.full((L,), 0x5F3759DF, jnp.int32)
  rstep = jnp.full((L,), RSTEP, jnp.float32)

  def chunk_body(c, _):
    base = c * L
    axv = ax_v[pl.ds(base, L)]
    ayv = ay_v[pl.ds(base, L)]
    azv = az_v[pl.ds(base, L)]
    occv = occ_v[pl.ds(base, L)]
    zlov = zlo_v[pl.ds(base, L)]
    zhiv = zhi_v[pl.ds(base, L)]
    ylov = ylo_v[pl.ds(base, L)]
    ycntv = ycnt_v[pl.ds(base, L)]

    for j in range(L):
      zlo_s = zlov[j]
      zhi_s = zhiv[j]

      @pl.when(jnp.logical_and(wid >= zlo_s, wid <= zhi_s))
      def _():
        n = base + j
        nv = jnp.full((L,), n, jnp.int32)
        axs = _splat(axv, j)
        ays = _splat(ayv, j)
        azs = _splat(azv, j)
        occs = _splat(occv, j)
        ylo_s = ylov[j]
        ycnt_s = ycntv[j]
        dzv = zf - azs
        cdz = g22 * dzv
        g12dz = g12 * dzv
        g02dz = g02 * dzv
        cdz2 = cdz * cdz

        def row_body(yi, _c):
          y = ylo_s + yi
          dyv = jnp.full((L,), y, jnp.int32).astype(jnp.float32) - ays
          cdy = g12dz + g11 * dyv
          cxy = g02dz + g01 * dyv
          cdy2 = cdy * cdy
          r = y * 2

          def do_half(hh, xf):
            dxv = xf - axs
            cdx = cxy + g00 * dxv
            d2 = cdx * cdx + cdy2 + cdz2
            m = d2 <= rmax2
            bits = plsc.bitcast(d2, jnp.int32)
            y0 = plsc.bitcast(magic - lax.shift_right_logical(bits, 1),
                              jnp.float32)
            hx = half * d2
            y0 = y0 * (three_half - hx * y0 * y0)
            y0 = y0 * (three_half - hx * y0 * y0)
            y0 = y0 * (three_half - hx * y0 * y0)
            dist = d2 * y0
            rad = dist / rstep
            il_raw = rad.astype(jnp.int32)
            wh = rad - il_raw.astype(jnp.float32)
            wl = 1.0 - wh
            il = jnp.clip(il_raw, 0, NRAD - 1)
            ih = jnp.minimum(il + 1, NRAD - 1)
            dl = plsc.load_gather(dens_v, [nv, il])
            dh = plsc.load_gather(dens_v, [nv, ih])
            dens = wl * dl + wh * dh
            contrib = jnp.where(m, occs * dens, 0.0)
            plsc.addupdate(plane_v.at[r + hh], contrib)

          do_half(0, xf0)
          do_half(1, xf1)
          return _c

        lax.fori_loop(0, ycnt_s, row_body, None)

    return _

  lax.fori_loop(0, NATOMS // L, chunk_body, None)

  pltpu.async_copy(plane_v, out_h.at[wid], sem).wait()


def kernel(coordinates, active, occupancies, lmax, radial_densities,
           grid_to_cartesian):
  del lmax
  dtype = jnp.float32
  coords = coordinates[0].astype(dtype)
  ax = coords[:, 0]
  ay = coords[:, 1]
  az = coords[:, 2]
  occ = (occupancies[0] * active[0].astype(dtype)).astype(dtype)
  dens = radial_densities[0].astype(dtype)

  g = grid_to_cartesian.astype(dtype)
  gv = jnp.zeros((16,), dtype).at[0].set(g[0, 0]).at[1].set(g[0, 1]) \
      .at[2].set(g[0, 2]).at[3].set(g[1, 1]).at[4].set(g[1, 2]) \
      .at[5].set(g[2, 2])

  pad = 1e-3
  zwid = jnp.abs(RMAX / g[2, 2]) * (1.0 + 1e-5) + pad
  ywid = (RMAX + jnp.abs(g[1, 2]) * zwid) / jnp.abs(g[1, 1]) * (1.0 + 1e-5) \
      + pad
  zlo = jnp.ceil(az - zwid).astype(jnp.int32)
  zhi = jnp.floor(az + zwid).astype(jnp.int32)
  ylo_f = jnp.maximum(jnp.ceil(ay - ywid), 0.0)
  yhi_f = jnp.minimum(jnp.floor(ay + ywid), GRID - 1.0)
  ylo = ylo_f.astype(jnp.int32)
  ycnt = jnp.maximum(yhi_f.astype(jnp.int32) - ylo + 1, 0)

  mesh = plsc.VectorSubcoreMesh(core_axis_name="c", subcore_axis_name="s")
  run = pl.kernel(
      _sc_body,
      out_type=jax.ShapeDtypeStruct((GRID, 2 * GRID, L), dtype),
      mesh=mesh,
      compiler_params=pltpu.CompilerParams(needs_layout_passes=False),
      scratch_types=[
          pltpu.VMEM((NATOMS,), dtype),
          pltpu.VMEM((NATOMS,), dtype),
          pltpu.VMEM((NATOMS,), dtype),
          pltpu.VMEM((NATOMS,), dtype),
          pltpu.VMEM((NATOMS,), jnp.int32),
          pltpu.VMEM((NATOMS,), jnp.int32),
          pltpu.VMEM((NATOMS,), jnp.int32),
          pltpu.VMEM((NATOMS,), jnp.int32),
          pltpu.VMEM((NATOMS, NRAD), dtype),
          pltpu.VMEM((L,), dtype),
          pltpu.VMEM((2 * GRID, L), dtype),
          pltpu.SemaphoreType.DMA,
      ],
  )
  out = run(ax, ay, az, occ, zlo, zhi, ylo, ycnt, dens, gv)
  return out.reshape((1, GRID, GRID, GRID))
